# baseline (speedup 1.0000x reference)
import functools

import jax
import jax.numpy as jnp
from jax import lax
from jax.experimental import pallas as pl
from jax.experimental.pallas import tpu as pltpu
from jax.experimental.pallas import tpu_sc as plsc

CPT = 4
ECH = 2000
LANES = 16
UNROLL = 8

_MESH = plsc.VectorSubcoreMesh(core_axis_name="c", subcore_axis_name="s")
_SC_PARAMS = pltpu.CompilerParams(needs_layout_passes=False)


@functools.lru_cache(maxsize=None)
def _make_deg(N, E):
    ept = E // 32
    nchunk = ept // ECH
    nvec = ECH // LANES

    @functools.partial(
        pl.kernel,
        out_type=jax.ShapeDtypeStruct((32, N), jnp.float32),
        mesh=_MESH,
        compiler_params=_SC_PARAMS,
        scratch_types=[
            pltpu.VMEM((N,), jnp.float32),
            pltpu.VMEM((ECH,), jnp.int32),
        ],
    )
    def deg_kernel(dst_hbm, zeros_hbm, degp_hbm, acc, dv):
        c = lax.axis_index("c")
        t = lax.axis_index("s")
        w = t * 2 + c
        pltpu.sync_copy(zeros_hbm, acc)
        ones = jnp.ones((LANES,), jnp.float32)

        def chunk(jc, carry):
            pltpu.sync_copy(dst_hbm.at[pl.ds(w * ept + jc * ECH, ECH)], dv)

            def vec(i, cc):
                didx = dv[pl.ds(i * LANES, LANES)]
                plsc.addupdate_scatter(acc, [didx], ones)
                return cc

            lax.fori_loop(0, nvec, vec, 0)
            return carry

        lax.fori_loop(0, nchunk, chunk, 0)
        pltpu.sync_copy(acc, degp_hbm.at[w])

    return deg_kernel


@functools.lru_cache(maxsize=None)
def _make_spmm(N, E, H):
    nchunk = E // ECH
    nvec = ECH // LANES

    @functools.partial(
        pl.kernel,
        out_type=jax.ShapeDtypeStruct((H * N,), jnp.float32),
        mesh=_MESH,
        compiler_params=_SC_PARAMS,
        scratch_types=[
            pltpu.VMEM((CPT * N,), jnp.float32),
            pltpu.VMEM((2 * CPT * N,), jnp.float32),
            pltpu.VMEM((ECH,), jnp.int32),
            pltpu.VMEM((ECH,), jnp.int32),
        ],
    )
    def spmm_kernel(u_hbm, src_hbm, dst_hbm, zeros_hbm, s_hbm,
                    ut, acc, sv, dv):
        c = lax.axis_index("c")
        t = lax.axis_index("s")
        w = t * 2 + c
        pltpu.sync_copy(u_hbm.at[pl.ds(w * CPT * N, CPT * N)], ut)
        pltpu.sync_copy(zeros_hbm, acc.at[pl.ds(0, CPT * N)])
        pltpu.sync_copy(zeros_hbm, acc.at[pl.ds(CPT * N, CPT * N)])

        def chunk(jc, carry):
            pltpu.sync_copy(src_hbm.at[pl.ds(jc * ECH, ECH)], sv)
            pltpu.sync_copy(dst_hbm.at[pl.ds(jc * ECH, ECH)], dv)

            def vec(i, cc):
                sidx = sv[pl.ds(i * LANES, LANES)]
                didx = dv[pl.ds(i * LANES, LANES)]
                par = (lax.rem(i, 2) * (CPT * N)).astype(jnp.int32)
                pvec = jnp.broadcast_to(par, (LANES,))
                for k in range(CPT):
                    off = jnp.full((LANES,), k * N, jnp.int32)
                    vals = plsc.load_gather(ut, [sidx + off])
                    plsc.addupdate_scatter(acc, [didx + off + pvec], vals)
                return cc

            lax.fori_loop(0, nvec, vec, 0, unroll=UNROLL)
            return carry

        lax.fori_loop(0, nchunk, chunk, 0)

        def merge(i, cc):
            a = acc[pl.ds(i * LANES, LANES)]
            b = acc[pl.ds(CPT * N + i * LANES, LANES)]
            acc[pl.ds(i * LANES, LANES)] = a + b
            return cc

        lax.fori_loop(0, CPT * N // LANES, merge, 0, unroll=8)
        pltpu.sync_copy(acc.at[pl.ds(0, CPT * N)],
                        s_hbm.at[pl.ds(w * CPT * N, CPT * N)])

    return spmm_kernel


def _tc_first(x, W, degp):
    N, DF = x.shape
    H = W.shape[1]

    def body(x_ref, w_ref, p_ref, u_ref, dis_ref):
        ones = jnp.ones((1, 32), jnp.float32)
        deg = lax.dot_general(ones, p_ref[...], (((1,), (0,)), ((), ())),
                              preferred_element_type=jnp.float32) + 1.0
        dis = lax.rsqrt(deg)
        yT = lax.dot_general(w_ref[...], x_ref[...], (((0,), (1,)), ((), ())),
                             preferred_element_type=jnp.float32)
        u_ref[...] = yT * dis
        dis_ref[...] = dis

    return pl.pallas_call(
        body,
        out_shape=[
            jax.ShapeDtypeStruct((H, N), jnp.float32),
            jax.ShapeDtypeStruct((1, N), jnp.float32),
        ],
    )(x, W, degp)


def _tc_mid(sT, uT, dis, bcol, W):
    H, N = sT.shape

    def body(s_ref, u_ref, dis_ref, b_ref, w_ref, o_ref):
        dis = dis_ref[...]
        hT = jnp.maximum((s_ref[...] + u_ref[...]) * dis + b_ref[...], 0.0)
        yT = lax.dot_general(w_ref[...], hT, (((0,), (0,)), ((), ())),
                             preferred_element_type=jnp.float32)
        o_ref[...] = yT * dis

    return pl.pallas_call(
        body,
        out_shape=jax.ShapeDtypeStruct((H, N), jnp.float32),
    )(sT, uT, dis, bcol, W)


def _tc_final(sT, uT, dis, bcol, batch2, Wl, blr, G):
    H, N = sT.shape
    C = Wl.shape[1]

    def body(s_ref, u_ref, dis_ref, b_ref, bt_ref, wl_ref, bl_ref, o_ref):
        dis = dis_ref[...]
        hT = jnp.maximum((s_ref[...] + u_ref[...]) * dis + b_ref[...], 0.0)
        oneh = (bt_ref[...] == lax.broadcasted_iota(
            jnp.int32, (N, G), 1)).astype(jnp.float32)
        accT = lax.dot_general(hT, oneh, (((1,), (0,)), ((), ())),
                               preferred_element_type=jnp.float32)
        cnt = lax.dot_general(
            jnp.ones((1, N), jnp.float32), oneh, (((1,), (0,)), ((), ())),
            preferred_element_type=jnp.float32)
        pooledT = accT / jnp.maximum(cnt, 1.0)
        o_ref[...] = lax.dot_general(
            pooledT, wl_ref[...], (((0,), (0,)), ((), ())),
            preferred_element_type=jnp.float32) + bl_ref[...]

    return pl.pallas_call(
        body,
        out_shape=jax.ShapeDtypeStruct((G, C), jnp.float32),
    )(sT, uT, dis, bcol, batch2, Wl, blr)


def kernel(x, edge_index, batch, W0, b0, W1, b1, W2, b2, Wl, bl):
    N, DF = x.shape
    E = edge_index.shape[1]
    H = W0.shape[1]
    C = Wl.shape[1]
    G = 64

    src = edge_index[0].astype(jnp.int32)
    dst = edge_index[1].astype(jnp.int32)
    batch2 = batch.astype(jnp.int32).reshape(N, 1)
    b0c = b0.reshape(H, 1)
    b1c = b1.reshape(H, 1)
    b2c = b2.reshape(H, 1)
    blr = bl.reshape(1, C)

    zerosN = jnp.zeros((N,), jnp.float32)
    zeros4N = jnp.zeros((CPT * N,), jnp.float32)

    deg = _make_deg(N, E)
    spmm = _make_spmm(N, E, H)

    degp = deg(dst, zerosN)
    uT0, dis = _tc_first(x, W0, degp)
    s0 = spmm(uT0.reshape(-1), src, dst, zeros4N).reshape(H, N)
    uT1 = _tc_mid(s0, uT0, dis, b0c, W1)
    s1 = spmm(uT1.reshape(-1), src, dst, zeros4N).reshape(H, N)
    uT2 = _tc_mid(s1, uT1, dis, b1c, W2)
    s2 = spmm(uT2.reshape(-1), src, dst, zeros4N).reshape(H, N)
    return _tc_final(s2, uT2, dis, b2c, batch2, Wl, blr, G)


# speedup vs baseline: 1.3753x; 1.3753x over previous
import functools

import jax
import jax.numpy as jnp
from jax import lax
from jax.experimental import pallas as pl
from jax.experimental.pallas import tpu as pltpu
from jax.experimental.pallas import tpu_sc as plsc

CPT = 4
ECH = 1600
DECH = 2000
LANES = 16
UNROLL = 4

_MESH = plsc.VectorSubcoreMesh(core_axis_name="c", subcore_axis_name="s")
_SC_PARAMS = pltpu.CompilerParams(needs_layout_passes=False)


@functools.lru_cache(maxsize=None)
def _make_deg(N, E):
    ept = E // 32
    nchunk = ept // DECH
    nvec = DECH // LANES

    @functools.partial(
        pl.kernel,
        out_type=jax.ShapeDtypeStruct((32, N), jnp.float32),
        mesh=_MESH,
        compiler_params=_SC_PARAMS,
        scratch_types=[
            pltpu.VMEM((N,), jnp.float32),
            pltpu.VMEM((N,), jnp.float32),
            pltpu.VMEM((DECH,), jnp.int32),
        ],
    )
    def deg_kernel(dst_hbm, degp_hbm, acc_e, acc_o, dv):
        c = lax.axis_index("c")
        t = lax.axis_index("s")
        w = t * 2 + c
        zeros = jnp.zeros((LANES,), jnp.float32)
        ones = jnp.ones((LANES,), jnp.float32)

        def zbody(i, cc):
            acc_e[pl.ds(i * LANES, LANES)] = zeros
            acc_o[pl.ds(i * LANES, LANES)] = zeros
            return cc

        lax.fori_loop(0, N // LANES, zbody, 0, unroll=8)

        def one(i, accx):
            didx = dv[pl.ds(i * LANES, LANES)]
            plsc.addupdate_scatter(accx, [didx], ones)

        def chunk(jc, carry):
            pltpu.sync_copy(dst_hbm.at[pl.ds(w * ept + jc * DECH, DECH)], dv)

            def pair(p, cc):
                one(p * 2, acc_e)
                one(p * 2 + 1, acc_o)
                return cc

            lax.fori_loop(0, nvec // 2, pair, 0, unroll=UNROLL)
            if nvec % 2:
                one(nvec - 1, acc_e)
            return carry

        lax.fori_loop(0, nchunk, chunk, 0)

        def merge(i, cc):
            acc_e[pl.ds(i * LANES, LANES)] = (
                acc_e[pl.ds(i * LANES, LANES)]
                + acc_o[pl.ds(i * LANES, LANES)])
            return cc

        lax.fori_loop(0, N // LANES, merge, 0, unroll=8)
        pltpu.sync_copy(acc_e, degp_hbm.at[w])

    return deg_kernel


@functools.lru_cache(maxsize=None)
def _make_spmm(N, E, H):
    nchunk = E // ECH
    nvec = ECH // LANES

    @functools.partial(
        pl.kernel,
        out_type=jax.ShapeDtypeStruct((H * N,), jnp.float32),
        mesh=_MESH,
        compiler_params=_SC_PARAMS,
        scratch_types=[
            pltpu.VMEM((CPT * N,), jnp.float32),
            pltpu.VMEM((CPT * N,), jnp.float32),
            pltpu.VMEM((CPT * N,), jnp.float32),
            pltpu.VMEM((ECH,), jnp.int32),
            pltpu.VMEM((ECH,), jnp.int32),
            pltpu.VMEM((ECH,), jnp.int32),
            pltpu.VMEM((ECH,), jnp.int32),
            pltpu.SemaphoreType.DMA,
            pltpu.SemaphoreType.DMA,
            pltpu.SemaphoreType.DMA,
        ],
    )
    def spmm_kernel(u_hbm, src_hbm, dst_hbm, s_hbm,
                    ut, acc_e, acc_o, sva, dva, svb, dvb, sema, semb, semu):
        c = lax.axis_index("c")
        t = lax.axis_index("s")
        w = t * 2 + c

        def start(jc, svx, dvx, sem):
            pltpu.async_copy(src_hbm.at[pl.ds(jc * ECH, ECH)], svx, sem)
            pltpu.async_copy(dst_hbm.at[pl.ds(jc * ECH, ECH)], dvx, sem)

        def wait(jc, svx, dvx, sem):
            pltpu.make_async_copy(
                src_hbm.at[pl.ds(jc * ECH, ECH)], svx, sem).wait()
            pltpu.make_async_copy(
                dst_hbm.at[pl.ds(jc * ECH, ECH)], dvx, sem).wait()

        pltpu.async_copy(u_hbm.at[pl.ds(w * CPT * N, CPT * N)], ut, semu)
        start(0, sva, dva, sema)

        zeros = jnp.zeros((LANES,), jnp.float32)

        def zbody(i, cc):
            acc_e[pl.ds(i * LANES, LANES)] = zeros
            acc_o[pl.ds(i * LANES, LANES)] = zeros
            return cc

        lax.fori_loop(0, CPT * N // LANES, zbody, 0, unroll=8)
        pltpu.make_async_copy(
            u_hbm.at[pl.ds(w * CPT * N, CPT * N)], ut, semu).wait()

        def one(svx, dvx, i, accx):
            sidx = svx[pl.ds(i * LANES, LANES)]
            didx = dvx[pl.ds(i * LANES, LANES)]
            for k in range(CPT):
                off = jnp.full((LANES,), k * N, jnp.int32)
                vals = plsc.load_gather(ut, [sidx + off])
                plsc.addupdate_scatter(accx, [didx + off], vals)

        def compute(svx, dvx):
            def pair(p, cc):
                one(svx, dvx, p * 2, acc_e)
                one(svx, dvx, p * 2 + 1, acc_o)
                return cc

            lax.fori_loop(0, nvec // 2, pair, 0, unroll=UNROLL)
            if nvec % 2:
                one(svx, dvx, nvec - 1, acc_e)

        def chunk2(j2, carry):
            jc0 = 2 * j2
            wait(jc0, sva, dva, sema)
            start(jc0 + 1, svb, dvb, semb)
            compute(sva, dva)
            wait(jc0 + 1, svb, dvb, semb)

            @pl.when(jc0 + 2 < nchunk)
            def _():
                start(jc0 + 2, sva, dva, sema)

            compute(svb, dvb)
            return carry

        lax.fori_loop(0, nchunk // 2, chunk2, 0)

        def merge(i, cc):
            acc_e[pl.ds(i * LANES, LANES)] = (
                acc_e[pl.ds(i * LANES, LANES)]
                + acc_o[pl.ds(i * LANES, LANES)])
            return cc

        lax.fori_loop(0, CPT * N // LANES, merge, 0, unroll=8)
        pltpu.sync_copy(acc_e, s_hbm.at[pl.ds(w * CPT * N, CPT * N)])

    return spmm_kernel


def _tc_first(x, W, degp):
    N, DF = x.shape
    H = W.shape[1]

    def body(x_ref, w_ref, p_ref, u_ref, dis_ref):
        ones = jnp.ones((1, 32), jnp.float32)
        deg = lax.dot_general(ones, p_ref[...], (((1,), (0,)), ((), ())),
                              preferred_element_type=jnp.float32) + 1.0
        dis = lax.rsqrt(deg)
        yT = lax.dot_general(w_ref[...], x_ref[...], (((0,), (1,)), ((), ())),
                             preferred_element_type=jnp.float32)
        u_ref[...] = yT * dis
        dis_ref[...] = dis

    return pl.pallas_call(
        body,
        out_shape=[
            jax.ShapeDtypeStruct((H, N), jnp.float32),
            jax.ShapeDtypeStruct((1, N), jnp.float32),
        ],
    )(x, W, degp)


def _tc_mid(sT, uT, dis, bcol, W):
    H, N = sT.shape

    def body(s_ref, u_ref, dis_ref, b_ref, w_ref, o_ref):
        dis = dis_ref[...]
        hT = jnp.maximum((s_ref[...] + u_ref[...]) * dis + b_ref[...], 0.0)
        yT = lax.dot_general(w_ref[...], hT, (((0,), (0,)), ((), ())),
                             preferred_element_type=jnp.float32)
        o_ref[...] = yT * dis

    return pl.pallas_call(
        body,
        out_shape=jax.ShapeDtypeStruct((H, N), jnp.float32),
    )(sT, uT, dis, bcol, W)


def _tc_final(sT, uT, dis, bcol, batch2, Wl, blr, G):
    H, N = sT.shape
    C = Wl.shape[1]

    def body(s_ref, u_ref, dis_ref, b_ref, bt_ref, wl_ref, bl_ref, o_ref):
        dis = dis_ref[...]
        hT = jnp.maximum((s_ref[...] + u_ref[...]) * dis + b_ref[...], 0.0)
        oneh = (bt_ref[...] == lax.broadcasted_iota(
            jnp.int32, (N, G), 1)).astype(jnp.float32)
        accT = lax.dot_general(hT, oneh, (((1,), (0,)), ((), ())),
                               preferred_element_type=jnp.float32)
        cnt = lax.dot_general(
            jnp.ones((1, N), jnp.float32), oneh, (((1,), (0,)), ((), ())),
            preferred_element_type=jnp.float32)
        pooledT = accT / jnp.maximum(cnt, 1.0)
        o_ref[...] = lax.dot_general(
            pooledT, wl_ref[...], (((0,), (0,)), ((), ())),
            preferred_element_type=jnp.float32) + bl_ref[...]

    return pl.pallas_call(
        body,
        out_shape=jax.ShapeDtypeStruct((G, C), jnp.float32),
    )(sT, uT, dis, bcol, batch2, Wl, blr)


def kernel(x, edge_index, batch, W0, b0, W1, b1, W2, b2, Wl, bl):
    N, DF = x.shape
    E = edge_index.shape[1]
    H = W0.shape[1]
    C = Wl.shape[1]
    G = 64

    src = edge_index[0].astype(jnp.int32)
    dst = edge_index[1].astype(jnp.int32)
    batch2 = batch.astype(jnp.int32).reshape(N, 1)
    b0c = b0.reshape(H, 1)
    b1c = b1.reshape(H, 1)
    b2c = b2.reshape(H, 1)
    blr = bl.reshape(1, C)

    deg = _make_deg(N, E)
    spmm = _make_spmm(N, E, H)

    degp = deg(dst)
    uT0, dis = _tc_first(x, W0, degp)
    s0 = spmm(uT0.reshape(-1), src, dst).reshape(H, N)
    uT1 = _tc_mid(s0, uT0, dis, b0c, W1)
    s1 = spmm(uT1.reshape(-1), src, dst).reshape(H, N)
    uT2 = _tc_mid(s1, uT1, dis, b1c, W2)
    s2 = spmm(uT2.reshape(-1), src, dst).reshape(H, N)
    return _tc_final(s2, uT2, dis, b2c, batch2, Wl, blr, G)
